# Initial kernel scaffold; baseline (speedup 1.0000x reference)
import jax
import jax.numpy as jnp
from jax.experimental import pallas as pl


def kernel(x, edge_index, W1, b1, W2, b2):
    raise NotImplementedError("write your pallas kernel here")


# speedup vs baseline: 14.7492x; 14.7492x over previous
import dataclasses
import functools

import jax
import jax.numpy as jnp
from jax import lax
from jax.experimental import pallas as pl
from jax.experimental.pallas import tpu as pltpu
from jax.experimental.pallas import tpu_sc as plsc

N_NODES = 10000
NP = 10112
F1 = 16
NC, NS = 2, 16
NW = NC * NS
CB = 128
STRIPE = NP // NS


def _sc_compiler_params():
    cp = pltpu.CompilerParams()
    fields = pltpu.CompilerParams.__dataclass_fields__
    if "needs_layout_passes" in fields:
        cp = dataclasses.replace(cp, needs_layout_passes=False)
    if "use_tc_tiling_on_sc" in fields:
        cp = dataclasses.replace(cp, use_tc_tiling_on_sc=False)
    return cp


def _pad_edges(ei):
    e = ei.shape[1]
    ep = -(-e // (NW * CB * 8)) * (NW * CB * 8)
    ei = jnp.concatenate(
        [ei, jnp.full((2, ep - e), N_NODES, jnp.int32)], axis=1)
    return ei.reshape(2, ep // CB, CB)


def _sc_degree(ei3):
    rows = ei3.shape[1]
    rt = rows // NW
    mesh = plsc.VectorSubcoreMesh(core_axis_name="c", subcore_axis_name="s")

    @functools.partial(
        pl.kernel,
        out_type=jax.ShapeDtypeStruct((NW, 1, NP), jnp.float32),
        mesh=mesh,
        scratch_types=[
            pltpu.VMEM((NP,), jnp.float32),
            pltpu.VMEM((rt, CB), jnp.int32),
            pltpu.VMEM((rt, CB), jnp.int32),
        ],
        compiler_params=_sc_compiler_params(),
    )
    def deg_kernel(ei_hbm, out_hbm, hist, esrc, edst):
        cid = lax.axis_index("c")
        sid = lax.axis_index("s")
        wid = cid * NS + sid

        @pl.loop(0, NP // 16)
        def _(i):
            hist[pl.ds(i * 16, 16)] = jnp.zeros((16,), jnp.float32)

        pltpu.sync_copy(ei_hbm.at[0, pl.ds(wid * rt, rt)], esrc)
        pltpu.sync_copy(ei_hbm.at[1, pl.ds(wid * rt, rt)], edst)

        ones = jnp.ones((16,), jnp.float32)

        @pl.loop(0, rt)
        def _(j):
            for buf in (esrc, edst):
                for k in range(CB // 16):
                    idx = buf[j, pl.ds(k * 16, 16)]
                    plsc.addupdate_scatter(hist, [idx], ones)

        pltpu.sync_copy(hist, out_hbm.at[wid, 0])

    return deg_kernel(ei3)


def _sc_aggregate(y, ei3):
    rows = ei3.shape[1]
    rt = rows // NW
    mesh = plsc.VectorSubcoreMesh(core_axis_name="c", subcore_axis_name="s")

    @functools.partial(
        pl.kernel,
        out_type=jax.ShapeDtypeStruct((NC, NP, F1), jnp.float32),
        mesh=mesh,
        scratch_types=[
            pltpu.VMEM_SHARED((NP, F1), jnp.float32),
            pltpu.VMEM((rt, CB), jnp.int32),
            pltpu.VMEM((rt, CB), jnp.int32),
            pltpu.VMEM((CB, F1), jnp.float32),
            pltpu.VMEM((STRIPE, F1), jnp.float32),
            pltpu.SemaphoreType.DMA,
        ],
        compiler_params=_sc_compiler_params(),
    )
    def agg_kernel(y_hbm, ei_hbm, out_hbm, acc, esrc, edst, rows_v, zbuf, sem):
        cid = lax.axis_index("c")
        sid = lax.axis_index("s")
        wid = cid * NS + sid

        @pl.loop(0, STRIPE)
        def _(i):
            zbuf[i, :] = jnp.zeros((16,), jnp.float32)

        pltpu.sync_copy(zbuf, acc.at[pl.ds(sid * STRIPE, STRIPE)])
        plsc.subcore_barrier()

        pltpu.sync_copy(ei_hbm.at[0, pl.ds(wid * rt, rt)], esrc)
        pltpu.sync_copy(ei_hbm.at[1, pl.ds(wid * rt, rt)], edst)

        @pl.loop(0, rt)
        def _(j):
            pltpu.async_copy(y_hbm.at[esrc.at[j]], rows_v, sem).wait()
            pltpu.sync_copy(rows_v, acc.at[edst.at[j]], add=True)
            pltpu.async_copy(y_hbm.at[edst.at[j]], rows_v, sem).wait()
            pltpu.sync_copy(rows_v, acc.at[esrc.at[j]], add=True)

        plsc.subcore_barrier()
        pltpu.sync_copy(acc.at[pl.ds(sid * STRIPE, STRIPE)],
                        out_hbm.at[cid, pl.ds(sid * STRIPE, STRIPE)])

    return agg_kernel(y, ei3)


def _tc_project(x, w):
    n, kdim = x.shape
    f = w.shape[1]
    nb = 5
    bs = n // nb

    def body(x_ref, w_ref, o_ref):
        o_ref[...] = jnp.dot(x_ref[...], w_ref[...],
                             preferred_element_type=jnp.float32,
                             precision=lax.Precision.HIGHEST)

    return pl.pallas_call(
        body,
        grid=(nb,),
        in_specs=[pl.BlockSpec((bs, kdim), lambda i: (i, 0)),
                  pl.BlockSpec((kdim, f), lambda i: (0, 0))],
        out_specs=pl.BlockSpec((bs, f), lambda i: (i, 0)),
        out_shape=jax.ShapeDtypeStruct((n, f), jnp.float32),
    )(x, w)


def _tc_norm(partials):
    def body(p_ref, o_ref):
        deg = jnp.sum(p_ref[...], axis=0)
        o_ref[...] = lax.rsqrt(jnp.maximum(deg, 1.0))

    return pl.pallas_call(
        body, out_shape=jax.ShapeDtypeStruct((1, NP), jnp.float32),
    )(partials)


def _tc_scale_pad(u, norm_col):
    def body(u_ref, n_ref, o_ref):
        o_ref[pl.ds(0, N_NODES), :] = u_ref[...] * n_ref[pl.ds(0, N_NODES), :]
        o_ref[pl.ds(N_NODES, NP - N_NODES), :] = jnp.zeros(
            (NP - N_NODES, F1), jnp.float32)

    return pl.pallas_call(
        body, out_shape=jax.ShapeDtypeStruct((NP, F1), jnp.float32),
    )(u, norm_col)


def _tc_mid(parts, norm_col, b1):
    def body(p_ref, n_ref, b_ref, o_ref):
        agg = p_ref[0] + p_ref[1]
        nrm = n_ref[...]
        t = agg * nrm + b_ref[...]
        h = jnp.where(t > 0, t, jnp.exp(t) - 1.0)
        o_ref[...] = h * nrm

    return pl.pallas_call(
        body, out_shape=jax.ShapeDtypeStruct((NP, F1), jnp.float32),
    )(parts, norm_col, b1)


def _tc_final(parts, norm_col, w2, b2):
    f2 = w2.shape[1]

    def body(q_ref, n_ref, w_ref, b_ref, o_ref):
        agg = q_ref[0, pl.ds(0, N_NODES), :] + q_ref[1, pl.ds(0, N_NODES), :]
        agg = agg * n_ref[pl.ds(0, N_NODES), :]
        o_ref[...] = jnp.dot(agg, w_ref[...],
                             preferred_element_type=jnp.float32,
                             precision=lax.Precision.HIGHEST) + b_ref[...]

    return pl.pallas_call(
        body, out_shape=jax.ShapeDtypeStruct((N_NODES, f2), jnp.float32),
    )(parts, norm_col, w2, b2)


def kernel(x, edge_index, W1, b1, W2, b2):
    ei3 = _pad_edges(edge_index.astype(jnp.int32))
    partials = _sc_degree(ei3)
    u1 = _tc_project(x, W1)
    norm_row = _tc_norm(partials)
    norm_col = jnp.transpose(norm_row)
    y = _tc_scale_pad(u1, norm_col)
    p1 = _sc_aggregate(y, ei3)
    z = _tc_mid(p1, norm_col, jnp.reshape(b1, (1, F1)))
    p2 = _sc_aggregate(z, ei3)
    return _tc_final(p2, norm_col, W2, jnp.reshape(b2, (1, -1)))


# speedup vs baseline: 20.2628x; 1.3738x over previous
import dataclasses
import functools

import jax
import jax.numpy as jnp
from jax import lax
from jax.experimental import pallas as pl
from jax.experimental.pallas import tpu as pltpu
from jax.experimental.pallas import tpu_sc as plsc

N_NODES = 10000
NP = 10112
F1 = 16
NC, NS = 2, 16
NW = NC * NS
CB = 128
STRIPE = NP // NS


def _sc_compiler_params():
    cp = pltpu.CompilerParams()
    fields = pltpu.CompilerParams.__dataclass_fields__
    if "needs_layout_passes" in fields:
        cp = dataclasses.replace(cp, needs_layout_passes=False)
    if "use_tc_tiling_on_sc" in fields:
        cp = dataclasses.replace(cp, use_tc_tiling_on_sc=False)
    return cp


def _pad_edges(ei):
    e = ei.shape[1]
    ep = -(-e // (NW * CB * 8)) * (NW * CB * 8)
    ei = jnp.concatenate(
        [ei, jnp.full((2, ep - e), N_NODES, jnp.int32)], axis=1)
    return ei.reshape(2, ep // CB, CB)


def _sc_degree(ei3):
    rows = ei3.shape[1]
    rt = rows // NW
    mesh = plsc.VectorSubcoreMesh(core_axis_name="c", subcore_axis_name="s")

    @functools.partial(
        pl.kernel,
        out_type=jax.ShapeDtypeStruct((NW, 1, NP), jnp.float32),
        mesh=mesh,
        scratch_types=[
            pltpu.VMEM((NP,), jnp.float32),
            pltpu.VMEM((rt, CB), jnp.int32),
            pltpu.VMEM((rt, CB), jnp.int32),
        ],
        compiler_params=_sc_compiler_params(),
    )
    def deg_kernel(ei_hbm, out_hbm, hist, esrc, edst):
        cid = lax.axis_index("c")
        sid = lax.axis_index("s")
        wid = cid * NS + sid

        @pl.loop(0, NP // 16)
        def _(i):
            hist[pl.ds(i * 16, 16)] = jnp.zeros((16,), jnp.float32)

        pltpu.sync_copy(ei_hbm.at[0, pl.ds(wid * rt, rt)], esrc)
        pltpu.sync_copy(ei_hbm.at[1, pl.ds(wid * rt, rt)], edst)

        ones = jnp.ones((16,), jnp.float32)

        @pl.loop(0, rt)
        def _(j):
            for buf in (esrc, edst):
                for k in range(CB // 16):
                    idx = buf[j, pl.ds(k * 16, 16)]
                    plsc.addupdate_scatter(hist, [idx], ones)

        pltpu.sync_copy(hist, out_hbm.at[wid, 0])

    return deg_kernel(ei3)


def _sc_aggregate(y, ei3):
    rows = ei3.shape[1]
    rt = rows // NW
    mesh = plsc.VectorSubcoreMesh(core_axis_name="c", subcore_axis_name="s")

    @functools.partial(
        pl.kernel,
        out_type=jax.ShapeDtypeStruct((NC, NP, F1), jnp.float32),
        mesh=mesh,
        scratch_types=[
            pltpu.VMEM_SHARED((NP, F1), jnp.float32),
            pltpu.VMEM((rt, CB), jnp.int32),
            pltpu.VMEM((rt, CB), jnp.int32),
            pltpu.VMEM((CB, F1), jnp.float32),
            pltpu.VMEM((CB, F1), jnp.float32),
            pltpu.VMEM((CB, F1), jnp.float32),
            pltpu.VMEM((CB, F1), jnp.float32),
            pltpu.VMEM((STRIPE, F1), jnp.float32),
            pltpu.SemaphoreType.DMA,
            pltpu.SemaphoreType.DMA,
        ],
        compiler_params=_sc_compiler_params(),
    )
    def agg_kernel(y_hbm, ei_hbm, out_hbm, acc, esrc, edst,
                   ra0, rb0, ra1, rb1, zbuf, s0, s1):
        cid = lax.axis_index("c")
        sid = lax.axis_index("s")
        wid = cid * NS + sid

        @pl.loop(0, STRIPE)
        def _(i):
            zbuf[i, :] = jnp.zeros((16,), jnp.float32)

        pltpu.sync_copy(zbuf, acc.at[pl.ds(sid * STRIPE, STRIPE)])
        plsc.subcore_barrier()

        pltpu.sync_copy(ei_hbm.at[0, pl.ds(wid * rt, rt)], esrc)
        pltpu.sync_copy(ei_hbm.at[1, pl.ds(wid * rt, rt)], edst)

        slots = ((ra0, rb0, s0), (ra1, rb1, s1))

        for b in range(2):
            ra, rb, sem = slots[b]
            pltpu.async_copy(y_hbm.at[esrc.at[b]], ra, sem)
            pltpu.async_copy(y_hbm.at[edst.at[b]], rb, sem)

        @pl.loop(0, rt, step=2)
        def _(j):
            for b in range(2):
                ra, rb, sem = slots[b]
                jc = j + b
                pltpu.make_async_copy(y_hbm.at[esrc.at[jc]], ra, sem).wait()
                pltpu.make_async_copy(y_hbm.at[edst.at[jc]], rb, sem).wait()
                pltpu.sync_copy(ra, acc.at[edst.at[jc]], add=True)
                pltpu.sync_copy(rb, acc.at[esrc.at[jc]], add=True)

                @pl.when(jc + 2 < rt)
                def _():
                    pltpu.async_copy(y_hbm.at[esrc.at[jc + 2]], ra, sem)
                    pltpu.async_copy(y_hbm.at[edst.at[jc + 2]], rb, sem)

        plsc.subcore_barrier()
        pltpu.sync_copy(acc.at[pl.ds(sid * STRIPE, STRIPE)],
                        out_hbm.at[cid, pl.ds(sid * STRIPE, STRIPE)])

    return agg_kernel(y, ei3)


def _tc_project(x, w):
    n, kdim = x.shape
    f = w.shape[1]
    nb = 5
    bs = n // nb

    def body(x_ref, w_ref, o_ref):
        o_ref[...] = jnp.dot(x_ref[...], w_ref[...],
                             preferred_element_type=jnp.float32,
                             precision=lax.Precision.HIGHEST)

    return pl.pallas_call(
        body,
        grid=(nb,),
        in_specs=[pl.BlockSpec((bs, kdim), lambda i: (i, 0)),
                  pl.BlockSpec((kdim, f), lambda i: (0, 0))],
        out_specs=pl.BlockSpec((bs, f), lambda i: (i, 0)),
        out_shape=jax.ShapeDtypeStruct((n, f), jnp.float32),
    )(x, w)


def _tc_norm(partials):
    def body(p_ref, o_ref):
        deg = jnp.sum(p_ref[...], axis=0)
        o_ref[...] = lax.rsqrt(jnp.maximum(deg, 1.0))

    return pl.pallas_call(
        body, out_shape=jax.ShapeDtypeStruct((1, NP), jnp.float32),
    )(partials)


def _tc_scale_pad(u, norm_col):
    def body(u_ref, n_ref, o_ref):
        o_ref[pl.ds(0, N_NODES), :] = u_ref[...] * n_ref[pl.ds(0, N_NODES), :]
        o_ref[pl.ds(N_NODES, NP - N_NODES), :] = jnp.zeros(
            (NP - N_NODES, F1), jnp.float32)

    return pl.pallas_call(
        body, out_shape=jax.ShapeDtypeStruct((NP, F1), jnp.float32),
    )(u, norm_col)


def _tc_mid(parts, norm_col, b1):
    def body(p_ref, n_ref, b_ref, o_ref):
        agg = p_ref[0] + p_ref[1]
        nrm = n_ref[...]
        t = agg * nrm + b_ref[...]
        h = jnp.where(t > 0, t, jnp.exp(t) - 1.0)
        o_ref[...] = h * nrm

    return pl.pallas_call(
        body, out_shape=jax.ShapeDtypeStruct((NP, F1), jnp.float32),
    )(parts, norm_col, b1)


def _tc_final(parts, norm_col, w2, b2):
    f2 = w2.shape[1]

    def body(q_ref, n_ref, w_ref, b_ref, o_ref):
        agg = q_ref[0, pl.ds(0, N_NODES), :] + q_ref[1, pl.ds(0, N_NODES), :]
        agg = agg * n_ref[pl.ds(0, N_NODES), :]
        o_ref[...] = jnp.dot(agg, w_ref[...],
                             preferred_element_type=jnp.float32,
                             precision=lax.Precision.HIGHEST) + b_ref[...]

    return pl.pallas_call(
        body, out_shape=jax.ShapeDtypeStruct((N_NODES, f2), jnp.float32),
    )(parts, norm_col, w2, b2)


def kernel(x, edge_index, W1, b1, W2, b2):
    ei3 = _pad_edges(edge_index.astype(jnp.int32))
    partials = _sc_degree(ei3)
    u1 = _tc_project(x, W1)
    norm_row = _tc_norm(partials)
    norm_col = jnp.transpose(norm_row)
    y = _tc_scale_pad(u1, norm_col)
    p1 = _sc_aggregate(y, ei3)
    z = _tc_mid(p1, norm_col, jnp.reshape(b1, (1, F1)))
    p2 = _sc_aggregate(z, ei3)
    return _tc_final(p2, norm_col, W2, jnp.reshape(b2, (1, -1)))


# speedup vs baseline: 20.7593x; 1.0245x over previous
import dataclasses
import functools

import jax
import jax.numpy as jnp
from jax import lax
from jax.experimental import pallas as pl
from jax.experimental.pallas import tpu as pltpu
from jax.experimental.pallas import tpu_sc as plsc

N_NODES = 10000
NP = 10112
F1 = 16
NC, NS = 2, 16
NW = NC * NS
CB = 128
STRIPE = NP // NS


def _sc_compiler_params():
    cp = pltpu.CompilerParams()
    fields = pltpu.CompilerParams.__dataclass_fields__
    if "needs_layout_passes" in fields:
        cp = dataclasses.replace(cp, needs_layout_passes=False)
    if "use_tc_tiling_on_sc" in fields:
        cp = dataclasses.replace(cp, use_tc_tiling_on_sc=False)
    return cp


def _pad_edges(ei):
    e = ei.shape[1]
    ep = -(-e // (NW * CB * 8)) * (NW * CB * 8)
    ei = jnp.concatenate(
        [ei, jnp.full((2, ep - e), N_NODES, jnp.int32)], axis=1)
    return ei.reshape(2, ep // CB, CB)


def _sc_degree(ei3):
    rows = ei3.shape[1]
    rt = rows // NW
    mesh = plsc.VectorSubcoreMesh(core_axis_name="c", subcore_axis_name="s")

    @functools.partial(
        pl.kernel,
        out_type=jax.ShapeDtypeStruct((NW, NP), jnp.float32),
        mesh=mesh,
        scratch_types=[
            pltpu.VMEM((NP,), jnp.float32),
            pltpu.VMEM((rt, CB), jnp.int32),
            pltpu.VMEM((rt, CB), jnp.int32),
        ],
        compiler_params=_sc_compiler_params(),
    )
    def deg_kernel(ei_hbm, out_hbm, hist, esrc, edst):
        cid = lax.axis_index("c")
        sid = lax.axis_index("s")
        wid = cid * NS + sid

        @pl.loop(0, NP // 16)
        def _(i):
            hist[pl.ds(i * 16, 16)] = jnp.zeros((16,), jnp.float32)

        pltpu.sync_copy(ei_hbm.at[0, pl.ds(wid * rt, rt)], esrc)
        pltpu.sync_copy(ei_hbm.at[1, pl.ds(wid * rt, rt)], edst)

        ones = jnp.ones((16,), jnp.float32)

        @pl.loop(0, rt)
        def _(j):
            for buf in (esrc, edst):
                for k in range(CB // 16):
                    idx = buf[j, pl.ds(k * 16, 16)]
                    plsc.addupdate_scatter(hist, [idx], ones)

        pltpu.sync_copy(hist, out_hbm.at[wid])

    return deg_kernel(ei3)


def _sc_aggregate(y, ei3):
    rows = ei3.shape[1]
    rt = rows // NW
    mesh = plsc.VectorSubcoreMesh(core_axis_name="c", subcore_axis_name="s")

    @functools.partial(
        pl.kernel,
        out_type=jax.ShapeDtypeStruct((NC, NP, F1), jnp.float32),
        mesh=mesh,
        scratch_types=[
            pltpu.VMEM_SHARED((NP, F1), jnp.float32),
            pltpu.VMEM((rt, CB), jnp.int32),
            pltpu.VMEM((rt, CB), jnp.int32),
            pltpu.VMEM((CB, F1), jnp.float32),
            pltpu.VMEM((CB, F1), jnp.float32),
            pltpu.VMEM((CB, F1), jnp.float32),
            pltpu.VMEM((CB, F1), jnp.float32),
            pltpu.VMEM((CB, F1), jnp.float32),
            pltpu.VMEM((CB, F1), jnp.float32),
            pltpu.VMEM((CB, F1), jnp.float32),
            pltpu.VMEM((CB, F1), jnp.float32),
            pltpu.VMEM((STRIPE, F1), jnp.float32),
            pltpu.SemaphoreType.DMA,
            pltpu.SemaphoreType.DMA,
            pltpu.SemaphoreType.DMA,
            pltpu.SemaphoreType.DMA,
        ],
        compiler_params=_sc_compiler_params(),
    )
    def agg_kernel(y_hbm, ei_hbm, out_hbm, acc, esrc, edst,
                   ra0, rb0, ra1, rb1, ra2, rb2, ra3, rb3,
                   zbuf, s0, s1, s2, s3):
        cid = lax.axis_index("c")
        sid = lax.axis_index("s")
        wid = cid * NS + sid

        @pl.loop(0, STRIPE)
        def _(i):
            zbuf[i, :] = jnp.zeros((16,), jnp.float32)

        pltpu.sync_copy(zbuf, acc.at[pl.ds(sid * STRIPE, STRIPE)])
        plsc.subcore_barrier()

        pltpu.sync_copy(ei_hbm.at[0, pl.ds(wid * rt, rt)], esrc)
        pltpu.sync_copy(ei_hbm.at[1, pl.ds(wid * rt, rt)], edst)

        slots = ((ra0, rb0, s0), (ra1, rb1, s1),
                 (ra2, rb2, s2), (ra3, rb3, s3))
        nbuf = len(slots)

        for b in range(nbuf):
            ra, rb, sem = slots[b]
            pltpu.async_copy(y_hbm.at[esrc.at[b]], ra, sem)
            pltpu.async_copy(y_hbm.at[edst.at[b]], rb, sem)

        @pl.loop(0, rt, step=nbuf)
        def _(j):
            for b in range(nbuf):
                ra, rb, sem = slots[b]
                jc = j + b
                pltpu.make_async_copy(y_hbm.at[esrc.at[jc]], ra, sem).wait()
                pltpu.make_async_copy(y_hbm.at[edst.at[jc]], rb, sem).wait()
                pltpu.sync_copy(ra, acc.at[edst.at[jc]], add=True)
                pltpu.sync_copy(rb, acc.at[esrc.at[jc]], add=True)

                @pl.when(jc + nbuf < rt)
                def _():
                    pltpu.async_copy(y_hbm.at[esrc.at[jc + nbuf]], ra, sem)
                    pltpu.async_copy(y_hbm.at[edst.at[jc + nbuf]], rb, sem)

        plsc.subcore_barrier()
        pltpu.sync_copy(acc.at[pl.ds(sid * STRIPE, STRIPE)],
                        out_hbm.at[cid, pl.ds(sid * STRIPE, STRIPE)])

    return agg_kernel(y, ei3)


def _tc_project(x, w):
    n, kdim = x.shape
    f = w.shape[1]
    nb = 5
    bs = n // nb

    def body(x_ref, w_ref, o_ref):
        o_ref[...] = jnp.dot(x_ref[...], w_ref[...],
                             preferred_element_type=jnp.float32,
                             precision=lax.Precision.HIGHEST)

    return pl.pallas_call(
        body,
        grid=(nb,),
        in_specs=[pl.BlockSpec((bs, kdim), lambda i: (i, 0)),
                  pl.BlockSpec((kdim, f), lambda i: (0, 0))],
        out_specs=pl.BlockSpec((bs, f), lambda i: (i, 0)),
        out_shape=jax.ShapeDtypeStruct((n, f), jnp.float32),
    )(x, w)


def _tc_norm_scale(partials, u):
    def body(p_ref, u_ref, y_ref, n_ref):
        ones = jnp.ones((NW, 1), jnp.float32)
        deg = lax.dot_general(p_ref[...], ones, (((0,), (0,)), ((), ())),
                              preferred_element_type=jnp.float32)
        nc = lax.rsqrt(jnp.maximum(deg, 1.0))
        n_ref[...] = nc
        y_ref[pl.ds(0, N_NODES), :] = u_ref[...] * nc[:N_NODES, :]
        y_ref[pl.ds(N_NODES, NP - N_NODES), :] = jnp.zeros(
            (NP - N_NODES, F1), jnp.float32)

    return pl.pallas_call(
        body,
        out_shape=(jax.ShapeDtypeStruct((NP, F1), jnp.float32),
                   jax.ShapeDtypeStruct((NP, 1), jnp.float32)),
    )(partials, u)


def _tc_mid(parts, norm_col, b1):
    def body(p_ref, n_ref, b_ref, o_ref):
        agg = p_ref[0] + p_ref[1]
        nrm = n_ref[...]
        t = agg * nrm + b_ref[...]
        h = jnp.where(t > 0, t, jnp.exp(t) - 1.0)
        o_ref[...] = h * nrm

    return pl.pallas_call(
        body, out_shape=jax.ShapeDtypeStruct((NP, F1), jnp.float32),
    )(parts, norm_col, b1)


def _tc_final(parts, norm_col, w2, b2):
    f2 = w2.shape[1]

    def body(q_ref, n_ref, w_ref, b_ref, o_ref):
        agg = q_ref[0, pl.ds(0, N_NODES), :] + q_ref[1, pl.ds(0, N_NODES), :]
        agg = agg * n_ref[pl.ds(0, N_NODES), :]
        o_ref[...] = jnp.dot(agg, w_ref[...],
                             preferred_element_type=jnp.float32,
                             precision=lax.Precision.HIGHEST) + b_ref[...]

    return pl.pallas_call(
        body, out_shape=jax.ShapeDtypeStruct((N_NODES, f2), jnp.float32),
    )(parts, norm_col, w2, b2)


def kernel(x, edge_index, W1, b1, W2, b2):
    ei3 = _pad_edges(edge_index.astype(jnp.int32))
    partials = _sc_degree(ei3)
    u1 = _tc_project(x, W1)
    y, norm_col = _tc_norm_scale(partials, u1)
    p1 = _sc_aggregate(y, ei3)
    z = _tc_mid(p1, norm_col, jnp.reshape(b1, (1, F1)))
    p2 = _sc_aggregate(z, ei3)
    return _tc_final(p2, norm_col, W2, jnp.reshape(b2, (1, -1)))


# speedup vs baseline: 31.2668x; 1.5062x over previous
import dataclasses
import functools

import jax
import jax.numpy as jnp
from jax import lax
from jax.experimental import pallas as pl
from jax.experimental.pallas import tpu as pltpu
from jax.experimental.pallas import tpu_sc as plsc

N_NODES = 10000
NP = 10112
F1 = 16
NC, NS = 2, 16
NW = NC * NS
CB = 128
STRIPE = NP // NS


def _sc_compiler_params():
    cp = pltpu.CompilerParams()
    fields = pltpu.CompilerParams.__dataclass_fields__
    if "needs_layout_passes" in fields:
        cp = dataclasses.replace(cp, needs_layout_passes=False)
    if "use_tc_tiling_on_sc" in fields:
        cp = dataclasses.replace(cp, use_tc_tiling_on_sc=False)
    return cp


def _pad_edges(ei):
    e = ei.shape[1]
    ep = -(-e // (NW * CB * 8)) * (NW * CB * 8)
    pad = N_NODES + (jnp.arange(ep - e, dtype=jnp.int32) % (NP - N_NODES))
    ei = jnp.concatenate([ei, jnp.stack([pad, pad])], axis=1)
    return ei.reshape(2, ep // CB, CB)


def _sc_degree(ei3):
    rows = ei3.shape[1]
    rt = rows // NW
    mesh = plsc.VectorSubcoreMesh(core_axis_name="c", subcore_axis_name="s")

    @functools.partial(
        pl.kernel,
        out_type=jax.ShapeDtypeStruct((NW, NP), jnp.float32),
        mesh=mesh,
        scratch_types=[
            pltpu.VMEM((NP,), jnp.float32),
            pltpu.VMEM((rt, CB), jnp.int32),
            pltpu.VMEM((rt, CB), jnp.int32),
        ],
        compiler_params=_sc_compiler_params(),
    )
    def deg_kernel(ei_hbm, out_hbm, hist, esrc, edst):
        cid = lax.axis_index("c")
        sid = lax.axis_index("s")
        wid = cid * NS + sid

        @pl.loop(0, NP // 16)
        def _(i):
            hist[pl.ds(i * 16, 16)] = jnp.zeros((16,), jnp.float32)

        pltpu.sync_copy(ei_hbm.at[0, pl.ds(wid * rt, rt)], esrc)
        pltpu.sync_copy(ei_hbm.at[1, pl.ds(wid * rt, rt)], edst)

        ones = jnp.ones((16,), jnp.float32)

        @pl.loop(0, rt)
        def _(j):
            for buf in (esrc, edst):
                for k in range(CB // 16):
                    idx = buf[j, pl.ds(k * 16, 16)]
                    plsc.addupdate_scatter(hist, [idx], ones)

        pltpu.sync_copy(hist, out_hbm.at[wid])

    return deg_kernel(ei3)


def _sc_aggregate(y, ei3):
    rows = ei3.shape[1]
    rt = rows // NW
    mesh = plsc.VectorSubcoreMesh(core_axis_name="c", subcore_axis_name="s")

    @functools.partial(
        pl.kernel,
        out_type=jax.ShapeDtypeStruct((NC, NP, F1), jnp.float32),
        mesh=mesh,
        scratch_types=[
            pltpu.VMEM_SHARED((NP, F1), jnp.float32),
            pltpu.VMEM((rt, CB), jnp.int32),
            pltpu.VMEM((rt, CB), jnp.int32),
            pltpu.VMEM((CB, F1), jnp.float32),
            pltpu.VMEM((CB, F1), jnp.float32),
            pltpu.VMEM((CB, F1), jnp.float32),
            pltpu.VMEM((CB, F1), jnp.float32),
            pltpu.VMEM((CB, F1), jnp.float32),
            pltpu.VMEM((CB, F1), jnp.float32),
            pltpu.VMEM((CB, F1), jnp.float32),
            pltpu.VMEM((CB, F1), jnp.float32),
            pltpu.VMEM((STRIPE, F1), jnp.float32),
            pltpu.SemaphoreType.DMA,
            pltpu.SemaphoreType.DMA,
            pltpu.SemaphoreType.DMA,
            pltpu.SemaphoreType.DMA,
        ],
        compiler_params=_sc_compiler_params(),
    )
    def agg_kernel(y_hbm, ei_hbm, out_hbm, acc, esrc, edst,
                   ra0, rb0, ra1, rb1, ra2, rb2, ra3, rb3,
                   zbuf, s0, s1, s2, s3):
        cid = lax.axis_index("c")
        sid = lax.axis_index("s")
        wid = cid * NS + sid

        @pl.loop(0, STRIPE)
        def _(i):
            zbuf[i, :] = jnp.zeros((16,), jnp.float32)

        pltpu.sync_copy(zbuf, acc.at[pl.ds(sid * STRIPE, STRIPE)])
        plsc.subcore_barrier()

        pltpu.sync_copy(ei_hbm.at[0, pl.ds(wid * rt, rt)], esrc)
        pltpu.sync_copy(ei_hbm.at[1, pl.ds(wid * rt, rt)], edst)

        slots = ((ra0, rb0, s0), (ra1, rb1, s1),
                 (ra2, rb2, s2), (ra3, rb3, s3))
        nbuf = len(slots)

        for b in range(nbuf):
            ra, rb, sem = slots[b]
            pltpu.async_copy(y_hbm.at[esrc.at[b]], ra, sem)
            pltpu.async_copy(y_hbm.at[edst.at[b]], rb, sem)

        @pl.loop(0, rt, step=nbuf)
        def _(j):
            for b in range(nbuf):
                ra, rb, sem = slots[b]
                jc = j + b
                pltpu.make_async_copy(y_hbm.at[esrc.at[jc]], ra, sem).wait()
                pltpu.make_async_copy(y_hbm.at[edst.at[jc]], rb, sem).wait()
                pltpu.sync_copy(ra, acc.at[edst.at[jc]], add=True)
                pltpu.sync_copy(rb, acc.at[esrc.at[jc]], add=True)

                @pl.when(jc + nbuf < rt)
                def _():
                    pltpu.async_copy(y_hbm.at[esrc.at[jc + nbuf]], ra, sem)
                    pltpu.async_copy(y_hbm.at[edst.at[jc + nbuf]], rb, sem)

        plsc.subcore_barrier()
        pltpu.sync_copy(acc.at[pl.ds(sid * STRIPE, STRIPE)],
                        out_hbm.at[cid, pl.ds(sid * STRIPE, STRIPE)])

    return agg_kernel(y, ei3)


def _tc_project(x, w):
    n, kdim = x.shape
    f = w.shape[1]
    nb = 5
    bs = n // nb

    def body(x_ref, w_ref, o_ref):
        o_ref[...] = jnp.dot(x_ref[...], w_ref[...],
                             preferred_element_type=jnp.float32,
                             precision=lax.Precision.HIGHEST)

    return pl.pallas_call(
        body,
        grid=(nb,),
        in_specs=[pl.BlockSpec((bs, kdim), lambda i: (i, 0)),
                  pl.BlockSpec((kdim, f), lambda i: (0, 0))],
        out_specs=pl.BlockSpec((bs, f), lambda i: (i, 0)),
        out_shape=jax.ShapeDtypeStruct((n, f), jnp.float32),
    )(x, w)


def _tc_norm_scale(partials, u):
    def body(p_ref, u_ref, y_ref, n_ref):
        ones = jnp.ones((NW, 1), jnp.float32)
        deg = lax.dot_general(p_ref[...], ones, (((0,), (0,)), ((), ())),
                              preferred_element_type=jnp.float32)
        nc = lax.rsqrt(jnp.maximum(deg, 1.0))
        n_ref[...] = nc
        y_ref[pl.ds(0, N_NODES), :] = u_ref[...] * nc[:N_NODES, :]
        y_ref[pl.ds(N_NODES, NP - N_NODES), :] = jnp.zeros(
            (NP - N_NODES, F1), jnp.float32)

    return pl.pallas_call(
        body,
        out_shape=(jax.ShapeDtypeStruct((NP, F1), jnp.float32),
                   jax.ShapeDtypeStruct((NP, 1), jnp.float32)),
    )(partials, u)


def _tc_mid(parts, norm_col, b1):
    def body(p_ref, n_ref, b_ref, o_ref):
        agg = p_ref[0] + p_ref[1]
        nrm = n_ref[...]
        t = agg * nrm + b_ref[...]
        h = jnp.where(t > 0, t, jnp.exp(t) - 1.0)
        o_ref[...] = h * nrm

    return pl.pallas_call(
        body, out_shape=jax.ShapeDtypeStruct((NP, F1), jnp.float32),
    )(parts, norm_col, b1)


def _tc_final(parts, norm_col, w2, b2):
    f2 = w2.shape[1]

    def body(q_ref, n_ref, w_ref, b_ref, o_ref):
        agg = q_ref[0, pl.ds(0, N_NODES), :] + q_ref[1, pl.ds(0, N_NODES), :]
        agg = agg * n_ref[pl.ds(0, N_NODES), :]
        o_ref[...] = jnp.dot(agg, w_ref[...],
                             preferred_element_type=jnp.float32,
                             precision=lax.Precision.HIGHEST) + b_ref[...]

    return pl.pallas_call(
        body, out_shape=jax.ShapeDtypeStruct((N_NODES, f2), jnp.float32),
    )(parts, norm_col, w2, b2)


def kernel(x, edge_index, W1, b1, W2, b2):
    ei3 = _pad_edges(edge_index.astype(jnp.int32))
    partials = _sc_degree(ei3)
    u1 = _tc_project(x, W1)
    y, norm_col = _tc_norm_scale(partials, u1)
    p1 = _sc_aggregate(y, ei3)
    z = _tc_mid(p1, norm_col, jnp.reshape(b1, (1, F1)))
    p2 = _sc_aggregate(z, ei3)
    return _tc_final(p2, norm_col, W2, jnp.reshape(b2, (1, -1)))
